# baseline (speedup 1.0000x reference)
import jax
import jax.numpy as jnp
from jax import lax
from jax.experimental import pallas as pl
from jax.experimental.pallas import tpu as pltpu
from jax.experimental.pallas import tpu_sc as plsc

N = 10000
D = 128
E = 160000

NS = 16
C = 96
NBC = 106
EP = NS * NBC * C
NPAD = 10128
RPT = 624


def _agg_body(x_hbm, src_hbm, dst_hbm, m_hbm, deg_hbm,
              acc, degsp, rows0, rows1, sidx0, didx0, sidx1, didx1,
              ones, zdeg, gsem0, gsem1):
    cid = lax.axis_index("c")
    sid = lax.axis_index("s")

    def _fill_row(i, _):
        for j in range(D // 16):
            rows0[i, pl.ds(j * 16, 16)] = jnp.zeros((16,), jnp.float32)
        return 0
    lax.fori_loop(0, C, _fill_row, 0)

    def _fill_ones(i, _):
        ones[pl.ds(i * 16, 16)] = jnp.ones((16,), jnp.float32)
        return 0
    lax.fori_loop(0, C // 16, _fill_ones, 0)

    def _fill_zdeg(i, _):
        zdeg[pl.ds(i * 16, 16)] = jnp.zeros((16,), jnp.float32)
        return 0
    lax.fori_loop(0, RPT // 16, _fill_zdeg, 0)

    rbase = sid * RPT
    for k in range(RPT // C):
        pltpu.sync_copy(rows0, acc.at[pl.ds(rbase + k * C, C), :])
    rem = RPT % C
    pltpu.sync_copy(rows0.at[pl.ds(0, rem), :],
                    acc.at[pl.ds(rbase + (RPT // C) * C, rem), :])
    pltpu.sync_copy(zdeg, degsp.at[pl.ds(rbase, RPT)])

    @pl.when(sid == NS - 1)
    def _():
        tail = NPAD - NS * RPT
        pltpu.sync_copy(rows0, acc.at[pl.ds(NS * RPT, C), :])
        pltpu.sync_copy(rows0.at[pl.ds(0, tail - C), :],
                        acc.at[pl.ds(NS * RPT + C, tail - C), :])
        pltpu.sync_copy(zdeg.at[pl.ds(0, tail)],
                        degsp.at[pl.ds(NS * RPT, tail)])

    plsc.subcore_barrier()

    ebase = cid * EP + sid * (NBC * C)

    def _fetch(c, sbuf, dbuf):
        off = ebase + c * C
        pltpu.sync_copy(src_hbm.at[pl.ds(off, C)], sbuf)
        pltpu.sync_copy(dst_hbm.at[pl.ds(off, C)], dbuf)

    _fetch(0, sidx0, didx0)
    pltpu.async_copy(x_hbm.at[sidx0], rows0, gsem0)

    def _blk(i, _):
        c0 = 2 * i
        _fetch(c0 + 1, sidx1, didx1)
        pltpu.async_copy(x_hbm.at[sidx1], rows1, gsem1)
        pltpu.make_async_copy(x_hbm.at[sidx0], rows0, gsem0).wait()
        pltpu.sync_copy(rows0, acc.at[didx0], add=True)
        pltpu.sync_copy(ones, degsp.at[didx0], add=True)

        @pl.when(i < NBC // 2 - 1)
        def _():
            _fetch(c0 + 2, sidx0, didx0)
            pltpu.async_copy(x_hbm.at[sidx0], rows0, gsem0)

        pltpu.make_async_copy(x_hbm.at[sidx1], rows1, gsem1).wait()
        pltpu.sync_copy(rows1, acc.at[didx1], add=True)
        pltpu.sync_copy(ones, degsp.at[didx1], add=True)
        return 0
    lax.fori_loop(0, NBC // 2, _blk, 0)

    plsc.subcore_barrier()

    pltpu.sync_copy(acc.at[pl.ds(rbase, RPT), :],
                    m_hbm.at[cid, pl.ds(rbase, RPT), :])
    pltpu.sync_copy(degsp.at[pl.ds(rbase, RPT)], zdeg)
    pltpu.sync_copy(zdeg, deg_hbm.at[pl.ds(cid * N + rbase, RPT)])

    @pl.when(sid == NS - 1)
    def _():
        tail = N - NS * RPT
        pltpu.sync_copy(acc.at[pl.ds(NS * RPT, tail), :],
                        m_hbm.at[cid, pl.ds(NS * RPT, tail), :])
        pltpu.sync_copy(degsp.at[pl.ds(NS * RPT, tail)],
                        zdeg.at[pl.ds(0, tail)])
        pltpu.sync_copy(zdeg.at[pl.ds(0, tail)],
                        deg_hbm.at[pl.ds(cid * N + NS * RPT, tail)])


_agg = pl.kernel(
    _agg_body,
    out_type=(
        jax.ShapeDtypeStruct((2, N, D), jnp.float32),
        jax.ShapeDtypeStruct((2 * N,), jnp.float32),
    ),
    mesh=plsc.VectorSubcoreMesh(core_axis_name="c", subcore_axis_name="s"),
    scratch_types=(
        pltpu.VMEM_SHARED((NPAD, D), jnp.float32),
        pltpu.VMEM_SHARED((NPAD,), jnp.float32),
        pltpu.VMEM((C, D), jnp.float32),
        pltpu.VMEM((C, D), jnp.float32),
        pltpu.VMEM((C,), jnp.int32),
        pltpu.VMEM((C,), jnp.int32),
        pltpu.VMEM((C,), jnp.int32),
        pltpu.VMEM((C,), jnp.int32),
        pltpu.VMEM((C,), jnp.float32),
        pltpu.VMEM((RPT,), jnp.float32),
        pltpu.SemaphoreType.DMA,
        pltpu.SemaphoreType.DMA,
    ),
)

BR = 1000


def _dense0_body(m0, m1, d0, d1, w0, w1, b0, b1, o):
    a0 = m0[...] * (1.0 / jnp.maximum(d0[...], 1.0))
    a1 = m1[...] * (1.0 / jnp.maximum(d1[...], 1.0))
    h = (jnp.dot(a0, w0[...], preferred_element_type=jnp.float32)
         + jnp.dot(a1, w1[...], preferred_element_type=jnp.float32))
    o[...] = jnp.maximum(h + b0[...] + b1[...], 0.0)


def _dense1_body(m0, m1, d0, d1, h0, w0, w1, ws, b0, b1, bs, o):
    a0 = m0[...] * (1.0 / jnp.maximum(d0[...], 1.0))
    a1 = m1[...] * (1.0 / jnp.maximum(d1[...], 1.0))
    h = (jnp.dot(a0, w0[...], preferred_element_type=jnp.float32)
         + jnp.dot(a1, w1[...], preferred_element_type=jnp.float32)
         + jnp.dot(h0[...], ws[...], preferred_element_type=jnp.float32))
    o[...] = h + b0[...] + b1[...] + bs[...]


def _row_spec():
    return pl.BlockSpec((BR, D), lambda i: (i, 0))


def _deg_spec():
    return pl.BlockSpec((BR, 1), lambda i: (i, 0))


def _mat_spec():
    return pl.BlockSpec((D, D), lambda i: (0, 0))


def _bias_spec():
    return pl.BlockSpec((1, D), lambda i: (0, 0))


_dense0 = pl.pallas_call(
    _dense0_body,
    grid=(N // BR,),
    in_specs=[_row_spec(), _row_spec(), _deg_spec(), _deg_spec(),
              _mat_spec(), _mat_spec(), _bias_spec(), _bias_spec()],
    out_specs=_row_spec(),
    out_shape=jax.ShapeDtypeStruct((N, D), jnp.float32),
)

_dense1 = pl.pallas_call(
    _dense1_body,
    grid=(N // BR,),
    in_specs=[_row_spec(), _row_spec(), _deg_spec(), _deg_spec(),
              _row_spec(), _mat_spec(), _mat_spec(), _mat_spec(),
              _bias_spec(), _bias_spec(), _bias_spec()],
    out_specs=_row_spec(),
    out_shape=jax.ShapeDtypeStruct((N, D), jnp.float32),
)


@jax.jit
def kernel(x, src_l0_e0, dst_l0_e0, src_l0_e1, dst_l0_e1,
           src_l1_e0, dst_l1_e0, src_l1_e1, dst_l1_e1,
           W_l0_e0, b_l0_e0, W_l0_e1, b_l0_e1,
           W_l1_e0, b_l1_e0, W_l1_e1, b_l1_e1,
           W_self, b_self):
    pad_s = jnp.zeros((EP - E,), jnp.int32)
    pad_d = N + (jnp.arange(EP - E, dtype=jnp.int32) % 128)
    src0 = jnp.concatenate([src_l0_e0, pad_s, src_l0_e1, pad_s])
    dst0 = jnp.concatenate([dst_l0_e0, pad_d, dst_l0_e1, pad_d])
    src1 = jnp.concatenate([src_l1_e0, pad_s, src_l1_e1, pad_s])
    dst1 = jnp.concatenate([dst_l1_e0, pad_d, dst_l1_e1, pad_d])

    m0, deg0 = _agg(x, src0, dst0)
    h0 = _dense0(m0[0], m0[1],
                 deg0[:N].reshape(N, 1), deg0[N:].reshape(N, 1),
                 W_l0_e0, W_l0_e1,
                 b_l0_e0.reshape(1, D), b_l0_e1.reshape(1, D))

    m1, deg1 = _agg(h0, src1, dst1)
    out = _dense1(m1[0], m1[1],
                  deg1[:N].reshape(N, 1), deg1[N:].reshape(N, 1),
                  h0, W_l1_e0, W_l1_e1, W_self,
                  b_l1_e0.reshape(1, D), b_l1_e1.reshape(1, D),
                  b_self.reshape(1, D))
    return out


# speedup vs baseline: 1.5987x; 1.5987x over previous
import jax
import jax.numpy as jnp
from jax import lax
from jax.experimental import pallas as pl
from jax.experimental.pallas import tpu as pltpu
from jax.experimental.pallas import tpu_sc as plsc

N = 10000
D = 128
E = 160000

NS = 16
C = 88
NBC = 114
EP = NS * NBC * C
NPAD = 10128
RPT = 624


def _agg_body(x_hbm, src_hbm, dst_hbm, m_hbm, deg_hbm,
              acc, degsp, rows0, rows1, sidx0, didx0, sidx1, didx1,
              ones, zdeg, gsem0, gsem1):
    cid = lax.axis_index("c")
    sid = lax.axis_index("s")

    def _fill_row(i, _):
        for j in range(D // 16):
            rows0[i, pl.ds(j * 16, 16)] = jnp.zeros((16,), jnp.float32)
        return 0
    lax.fori_loop(0, C, _fill_row, 0)

    def _fill_ones(i, _):
        ones[pl.ds(i * 16, 16)] = jnp.ones((16,), jnp.float32)
        return 0
    lax.fori_loop(0, C // 16, _fill_ones, 0)

    def _fill_zdeg(i, _):
        zdeg[pl.ds(i * 16, 16)] = jnp.zeros((16,), jnp.float32)
        return 0
    lax.fori_loop(0, RPT // 16, _fill_zdeg, 0)

    rbase = sid * RPT
    for k in range(RPT // C):
        pltpu.sync_copy(rows0, acc.at[pl.ds(rbase + k * C, C), :])
    rem = RPT % C
    pltpu.sync_copy(rows0.at[pl.ds(0, rem), :],
                    acc.at[pl.ds(rbase + (RPT // C) * C, rem), :])
    pltpu.sync_copy(zdeg, degsp.at[pl.ds(rbase, RPT)])

    @pl.when(sid == NS - 1)
    def _():
        tail = NPAD - NS * RPT
        pltpu.sync_copy(rows0, acc.at[pl.ds(NS * RPT, C), :])
        pltpu.sync_copy(rows0.at[pl.ds(0, tail - C), :],
                        acc.at[pl.ds(NS * RPT + C, tail - C), :])
        pltpu.sync_copy(zdeg.at[pl.ds(0, tail)],
                        degsp.at[pl.ds(NS * RPT, tail)])

    plsc.subcore_barrier()

    ebase = cid * EP + sid * (NBC * C)

    def _fetch(c, sbuf, dbuf):
        off = ebase + c * C
        pltpu.sync_copy(src_hbm.at[pl.ds(off, C)], sbuf)
        pltpu.sync_copy(dst_hbm.at[pl.ds(off, C)], dbuf)

    _fetch(0, sidx0, didx0)
    pltpu.async_copy(x_hbm.at[sidx0], rows0, gsem0)

    def _blk(i, _):
        c0 = 2 * i
        _fetch(c0 + 1, sidx1, didx1)
        pltpu.async_copy(x_hbm.at[sidx1], rows1, gsem1)
        pltpu.make_async_copy(x_hbm.at[sidx0], rows0, gsem0).wait()
        pltpu.sync_copy(rows0, acc.at[didx0], add=True)
        pltpu.sync_copy(ones, degsp.at[didx0], add=True)

        @pl.when(i < NBC // 2 - 1)
        def _():
            _fetch(c0 + 2, sidx0, didx0)
            pltpu.async_copy(x_hbm.at[sidx0], rows0, gsem0)

        pltpu.make_async_copy(x_hbm.at[sidx1], rows1, gsem1).wait()
        pltpu.sync_copy(rows1, acc.at[didx1], add=True)
        pltpu.sync_copy(ones, degsp.at[didx1], add=True)
        return 0
    lax.fori_loop(0, NBC // 2, _blk, 0)

    plsc.subcore_barrier()

    pltpu.sync_copy(acc.at[pl.ds(rbase, RPT), :],
                    m_hbm.at[cid, pl.ds(rbase, RPT), :])
    pltpu.sync_copy(degsp.at[pl.ds(rbase, RPT)], zdeg)
    pltpu.sync_copy(zdeg, deg_hbm.at[pl.ds(cid * N + rbase, RPT)])

    @pl.when(sid == NS - 1)
    def _():
        tail = N - NS * RPT
        pltpu.sync_copy(acc.at[pl.ds(NS * RPT, tail), :],
                        m_hbm.at[cid, pl.ds(NS * RPT, tail), :])
        pltpu.sync_copy(degsp.at[pl.ds(NS * RPT, tail)],
                        zdeg.at[pl.ds(0, tail)])
        pltpu.sync_copy(zdeg.at[pl.ds(0, tail)],
                        deg_hbm.at[pl.ds(cid * N + NS * RPT, tail)])


_agg = pl.kernel(
    _agg_body,
    out_type=(
        jax.ShapeDtypeStruct((2, N, D), jnp.float32),
        jax.ShapeDtypeStruct((2 * N,), jnp.float32),
    ),
    mesh=plsc.VectorSubcoreMesh(core_axis_name="c", subcore_axis_name="s"),
    scratch_types=(
        pltpu.VMEM_SHARED((NPAD, D), jnp.float32),
        pltpu.VMEM_SHARED((NPAD,), jnp.float32),
        pltpu.VMEM((C, D), jnp.float32),
        pltpu.VMEM((C, D), jnp.float32),
        pltpu.VMEM((C,), jnp.int32),
        pltpu.VMEM((C,), jnp.int32),
        pltpu.VMEM((C,), jnp.int32),
        pltpu.VMEM((C,), jnp.int32),
        pltpu.VMEM((C,), jnp.float32),
        pltpu.VMEM((RPT,), jnp.float32),
        pltpu.SemaphoreType.DMA,
        pltpu.SemaphoreType.DMA,
    ),
)

BR = 1000


def _dense0_body(m0, m1, d0, d1, w0, w1, b0, b1, o):
    a0 = m0[...] * (1.0 / jnp.maximum(d0[...], 1.0))
    a1 = m1[...] * (1.0 / jnp.maximum(d1[...], 1.0))
    h = (jnp.dot(a0, w0[...], preferred_element_type=jnp.float32)
         + jnp.dot(a1, w1[...], preferred_element_type=jnp.float32))
    o[...] = jnp.maximum(h + b0[...] + b1[...], 0.0)


def _dense1_body(m0, m1, d0, d1, h0, w0, w1, ws, b0, b1, bs, o):
    a0 = m0[...] * (1.0 / jnp.maximum(d0[...], 1.0))
    a1 = m1[...] * (1.0 / jnp.maximum(d1[...], 1.0))
    h = (jnp.dot(a0, w0[...], preferred_element_type=jnp.float32)
         + jnp.dot(a1, w1[...], preferred_element_type=jnp.float32)
         + jnp.dot(h0[...], ws[...], preferred_element_type=jnp.float32))
    o[...] = h + b0[...] + b1[...] + bs[...]


def _row_spec():
    return pl.BlockSpec((BR, D), lambda i: (i, 0))


def _deg_spec():
    return pl.BlockSpec((BR, 1), lambda i: (i, 0))


def _mat_spec():
    return pl.BlockSpec((D, D), lambda i: (0, 0))


def _bias_spec():
    return pl.BlockSpec((1, D), lambda i: (0, 0))


_dense0 = pl.pallas_call(
    _dense0_body,
    grid=(N // BR,),
    in_specs=[_row_spec(), _row_spec(), _deg_spec(), _deg_spec(),
              _mat_spec(), _mat_spec(), _bias_spec(), _bias_spec()],
    out_specs=_row_spec(),
    out_shape=jax.ShapeDtypeStruct((N, D), jnp.float32),
)

_dense1 = pl.pallas_call(
    _dense1_body,
    grid=(N // BR,),
    in_specs=[_row_spec(), _row_spec(), _deg_spec(), _deg_spec(),
              _row_spec(), _mat_spec(), _mat_spec(), _mat_spec(),
              _bias_spec(), _bias_spec(), _bias_spec()],
    out_specs=_row_spec(),
    out_shape=jax.ShapeDtypeStruct((N, D), jnp.float32),
)


@jax.jit
def kernel(x, src_l0_e0, dst_l0_e0, src_l0_e1, dst_l0_e1,
           src_l1_e0, dst_l1_e0, src_l1_e1, dst_l1_e1,
           W_l0_e0, b_l0_e0, W_l0_e1, b_l0_e1,
           W_l1_e0, b_l1_e0, W_l1_e1, b_l1_e1,
           W_self, b_self):
    pad_s = jnp.zeros((EP - E,), jnp.int32)
    pad_d = N + (jnp.arange(EP - E, dtype=jnp.int32) % 128)
    src0 = jnp.concatenate([src_l0_e0, pad_s, src_l0_e1, pad_s])
    dst0 = jnp.concatenate([dst_l0_e0, pad_d, dst_l0_e1, pad_d])
    src1 = jnp.concatenate([src_l1_e0, pad_s, src_l1_e1, pad_s])
    dst1 = jnp.concatenate([dst_l1_e0, pad_d, dst_l1_e1, pad_d])

    m0, deg0 = _agg(x, src0, dst0)
    h0 = _dense0(m0[0], m0[1],
                 deg0[:N].reshape(N, 1), deg0[N:].reshape(N, 1),
                 W_l0_e0, W_l0_e1,
                 b_l0_e0.reshape(1, D), b_l0_e1.reshape(1, D))

    m1, deg1 = _agg(h0, src1, dst1)
    out = _dense1(m1[0], m1[1],
                  deg1[:N].reshape(N, 1), deg1[N:].reshape(N, 1),
                  h0, W_l1_e0, W_l1_e1, W_self,
                  b_l1_e0.reshape(1, D), b_l1_e1.reshape(1, D),
                  b_self.reshape(1, D))
    return out
